# baseline (speedup 1.0000x reference)
import jax
import jax.numpy as jnp
from jax import lax
from jax.experimental import pallas as pl
from jax.experimental.pallas import tpu as pltpu

_K = 32
_NEG = -1e30
_CHUNKS = [(0, 80), (80, 80), (160, 80), (240, 80), (320, 80),
           (400, 80), (480, 80), (560, 17)]


def _body(awr_ref, awc_ref, fc_ref, fp_ref, edW_ref, edb_ref, euW_ref,
          eub_ref, ndW_ref, ndb_ref, nuW_ref, nub_ref, o_ref, SA, HEs):
    bf = jnp.bfloat16
    F = fc_ref[0]
    N = F.shape[0]
    Fp = fp_ref[0].astype(bf)

    sq = jnp.sum(F * F, axis=1, keepdims=True)
    inv = 1.0 / jnp.maximum(jnp.sqrt(sq), 1e-12)
    Fn = F * inv
    sim = lax.dot_general(Fn, Fn, (((1,), (1,)), ((), ())),
                          preferred_element_type=jnp.float32)
    row = lax.broadcasted_iota(jnp.int32, (N, N), 0)
    col = lax.broadcasted_iota(jnp.int32, (N, N), 1)
    aw_r = awr_ref[0]
    sim = jnp.where((row == col) | (col == 0), _NEG, sim)
    sim = jnp.where(row == 0, aw_r, sim)

    def he_chunk(c):
        start, size = _CHUNKS[c]
        A_c = SA[pl.ds(start, size), :]
        HE_c = jnp.dot(A_c, Fp, preferred_element_type=jnp.float32)
        h_c = jnp.dot(HE_c.astype(bf), edW_ref[...],
                      preferred_element_type=jnp.float32) + edb_ref[...]
        h_c = jnp.where(h_c >= 0, h_c, 0.2 * h_c)
        HEr_c = jnp.dot(h_c.astype(bf), euW_ref[...],
                        preferred_element_type=jnp.float32) + eub_ref[...]
        HEs[pl.ds(start, size), :] = HEr_c.astype(bf)

    work = sim
    m1 = None
    t = None
    for i in range(_K):
        m = jnp.max(work, axis=1, keepdims=True)
        if i == 0:
            m1 = m
        work = jnp.where(work >= m, _NEG, work)
        t = m
        if i % 4 == 3:
            he_chunk(i // 4)

    def hc_chunk(c):
        start, size = _CHUNKS[c]
        A_cT = SA[:, pl.ds(start, size)]
        HC_c = lax.dot_general(A_cT, HEs[...], (((0,), (0,)), ((), ())),
                               preferred_element_type=jnp.float32)
        h2_c = jnp.dot(HC_c.astype(bf), ndW_ref[...],
                       preferred_element_type=jnp.float32) + ndb_ref[...]
        h2_c = jnp.where(h2_c >= 0, h2_c, 0.2 * h2_c)
        out_c = jnp.dot(h2_c.astype(bf), nuW_ref[...],
                        preferred_element_type=jnp.float32) + nub_ref[...]
        o_ref[0, pl.ds(start, size), :] = out_c

    hc_chunk(0)
    e = jnp.where(sim >= t, jnp.exp(sim - m1), 0.0)
    hc_chunk(1)
    z = jnp.sum(e, axis=1, keepdims=True)
    hc_chunk(2)
    A = e / z
    hc_chunk(3)
    A = jnp.where(row == col, 1.0, A)
    hc_chunk(4)
    awc = awc_ref[0]
    t0 = t[0, 0]
    m0 = m1[0, 0]
    e0 = jnp.where(awc >= t0, jnp.exp(awc - m0), 0.0)
    r_col = e0 / jnp.sum(e0)
    rowc = lax.broadcasted_iota(jnp.int32, (N, 1), 0)
    a0c = jnp.where(rowc == 0, 1.0, r_col)
    hc_chunk(5)
    A = jnp.where(col == 0, a0c, A)
    hc_chunk(6)
    Ab = A.astype(bf)
    hc_chunk(7)
    SA[...] = Ab


def _build_call(B, N, D, H, interpret=False):
    lastb = B - 1
    return pl.pallas_call(
        _body,
        grid=(B + 1,),
        in_specs=[
            pl.BlockSpec((1, 1, N), lambda b: (jnp.minimum(b, lastb), 0, 0)),
            pl.BlockSpec((1, N, 1), lambda b: (jnp.minimum(b, lastb), 0, 0)),
            pl.BlockSpec((1, N, D), lambda b: (jnp.minimum(b, lastb), 0, 0)),
            pl.BlockSpec((1, N, D), lambda b: (jnp.maximum(b - 1, 0), 0, 0)),
            pl.BlockSpec((D, H), lambda b: (0, 0)),
            pl.BlockSpec((1, H), lambda b: (0, 0)),
            pl.BlockSpec((H, D), lambda b: (0, 0)),
            pl.BlockSpec((1, D), lambda b: (0, 0)),
            pl.BlockSpec((D, H), lambda b: (0, 0)),
            pl.BlockSpec((1, H), lambda b: (0, 0)),
            pl.BlockSpec((H, D), lambda b: (0, 0)),
            pl.BlockSpec((1, D), lambda b: (0, 0)),
        ],
        out_specs=pl.BlockSpec((1, N, D), lambda b: (jnp.maximum(b - 1, 0), 0, 0)),
        out_shape=jax.ShapeDtypeStruct((B, N, D), jnp.float32),
        scratch_shapes=[
            pltpu.VMEM((N, N), jnp.bfloat16),
            pltpu.VMEM((N, D), jnp.bfloat16),
        ],
        compiler_params=pltpu.CompilerParams(
            dimension_semantics=("arbitrary",)),
        interpret=interpret,
    )


def kernel(features, attn_weights, edge_down_W, edge_down_b, edge_up_W,
           edge_up_b, node_down_W, node_down_b, node_up_W, node_up_b):
    B, N, D = features.shape
    H = edge_down_W.shape[1]
    aw_pad = jnp.concatenate(
        [jnp.full((B, 1), _NEG, features.dtype), attn_weights], axis=1)
    call = _build_call(B, N, D, H)
    bf = jnp.bfloat16
    return call(aw_pad[:, None, :], aw_pad[:, :, None], features, features,
                edge_down_W.astype(bf), edge_down_b.reshape(1, -1),
                edge_up_W.astype(bf), edge_up_b.reshape(1, -1),
                node_down_W.astype(bf), node_down_b.reshape(1, -1),
                node_up_W.astype(bf), node_up_b.reshape(1, -1))


# speedup vs baseline: 1.0416x; 1.0416x over previous
import jax
import jax.numpy as jnp
from jax import lax
from jax.experimental import pallas as pl
from jax.experimental.pallas import tpu as pltpu

_K = 32
_NEG = -1e30


def _body(awr_ref, awc_ref, f_ref, edW_ref, edb_ref, euW_ref, eub_ref,
          ndW_ref, ndb_ref, nuW_ref, nub_ref, o_ref):
    F = f_ref[0]
    N = F.shape[0]

    sq = jnp.sum(F * F, axis=1, keepdims=True)
    inv = 1.0 / jnp.maximum(jnp.sqrt(sq), 1e-12)
    Fn = F * inv
    simT = lax.dot_general(Fn, Fn, (((1,), (1,)), ((), ())),
                           preferred_element_type=jnp.float32)
    row = lax.broadcasted_iota(jnp.int32, (N, N), 0)
    col = lax.broadcasted_iota(jnp.int32, (N, N), 1)
    awc = awc_ref[0]
    simT = jnp.where((row == col) | (row == 0), _NEG, simT)
    simT = jnp.where(col == 0, awc, simT)

    work = simT
    m1 = None
    t = None
    for i in range(_K):
        m = jnp.max(work, axis=0, keepdims=True)
        if i == 0:
            m1 = m
        work = jnp.where(work >= m, _NEG, work)
        t = m

    e = jnp.where(simT >= t, jnp.exp(simT - m1), 0.0)
    z = jnp.sum(e, axis=0, keepdims=True)
    AT = e / z
    AT = jnp.where(row == col, 1.0, AT)

    aw_r = awr_ref[0]
    t0 = t[0, 0]
    m0 = m1[0, 0]
    e0 = jnp.where(aw_r >= t0, jnp.exp(aw_r - m0), 0.0)
    r_row = e0 / jnp.sum(e0)
    r_row = jnp.where(lax.broadcasted_iota(jnp.int32, (1, N), 1) == 0,
                      1.0, r_row)
    AT = jnp.where(row == 0, r_row, AT)

    bf = jnp.bfloat16
    ATb = AT.astype(bf)
    HE = lax.dot_general(ATb, F.astype(bf), (((0,), (0,)), ((), ())),
                         preferred_element_type=jnp.float32)
    h = jnp.dot(HE.astype(bf), edW_ref[...], preferred_element_type=jnp.float32) + edb_ref[...]
    h = jnp.where(h >= 0, h, 0.2 * h)
    HEr = jnp.dot(h.astype(bf), euW_ref[...], preferred_element_type=jnp.float32) + eub_ref[...]
    HC = jnp.dot(ATb, HEr.astype(bf), preferred_element_type=jnp.float32)
    h2 = jnp.dot(HC.astype(bf), ndW_ref[...], preferred_element_type=jnp.float32) + ndb_ref[...]
    h2 = jnp.where(h2 >= 0, h2, 0.2 * h2)
    out = jnp.dot(h2.astype(bf), nuW_ref[...], preferred_element_type=jnp.float32) + nub_ref[...]
    o_ref[0] = out


def _build_call(B, N, D, H, interpret=False):
    return pl.pallas_call(
        _body,
        grid=(B,),
        in_specs=[
            pl.BlockSpec((1, 1, N), lambda b: (b, 0, 0)),
            pl.BlockSpec((1, N, 1), lambda b: (b, 0, 0)),
            pl.BlockSpec((1, N, D), lambda b: (b, 0, 0)),
            pl.BlockSpec((D, H), lambda b: (0, 0)),
            pl.BlockSpec((1, H), lambda b: (0, 0)),
            pl.BlockSpec((H, D), lambda b: (0, 0)),
            pl.BlockSpec((1, D), lambda b: (0, 0)),
            pl.BlockSpec((D, H), lambda b: (0, 0)),
            pl.BlockSpec((1, H), lambda b: (0, 0)),
            pl.BlockSpec((H, D), lambda b: (0, 0)),
            pl.BlockSpec((1, D), lambda b: (0, 0)),
        ],
        out_specs=pl.BlockSpec((1, N, D), lambda b: (b, 0, 0)),
        out_shape=jax.ShapeDtypeStruct((B, N, D), jnp.float32),
        compiler_params=pltpu.CompilerParams(
            dimension_semantics=("arbitrary",)),
        interpret=interpret,
    )


def kernel(features, attn_weights, edge_down_W, edge_down_b, edge_up_W,
           edge_up_b, node_down_W, node_down_b, node_up_W, node_up_b):
    B, N, D = features.shape
    H = edge_down_W.shape[1]
    aw_pad = jnp.concatenate(
        [jnp.full((B, 1), _NEG, features.dtype), attn_weights], axis=1)
    call = _build_call(B, N, D, H)
    bf = jnp.bfloat16
    return call(aw_pad[:, None, :], aw_pad[:, :, None], features,
                edge_down_W.astype(bf), edge_down_b.reshape(1, -1),
                edge_up_W.astype(bf), edge_up_b.reshape(1, -1),
                node_down_W.astype(bf), node_down_b.reshape(1, -1),
                node_up_W.astype(bf), node_up_b.reshape(1, -1))


# speedup vs baseline: 1.1928x; 1.1452x over previous
import jax
import jax.numpy as jnp
from jax import lax
from jax.experimental import pallas as pl
from jax.experimental.pallas import tpu as pltpu

_K = 32
_NEG = -1e30


def _body(aw_row_ref, aw_col_ref, f_ref, edW_ref, edb_ref, euW_ref, eub_ref,
          ndW_ref, ndb_ref, nuW_ref, nub_ref, o_ref):
    F = f_ref[0]
    N = F.shape[0]

    sq = jnp.sum(F * F, axis=1, keepdims=True)
    inv = 1.0 / jnp.maximum(jnp.sqrt(sq), 1e-12)
    Fn = F * inv
    sim = lax.dot_general(Fn, Fn, (((1,), (1,)), ((), ())),
                          preferred_element_type=jnp.float32)

    row = lax.broadcasted_iota(jnp.int32, (N, N), 0)
    col = lax.broadcasted_iota(jnp.int32, (N, N), 1)
    aw_r = aw_row_ref[0]
    sim = jnp.where((row == col) | (col == 0), _NEG, sim)
    sim = jnp.where(row == 0, aw_r, sim)

    t = jnp.max(sim, axis=1, keepdims=True)
    m1 = t
    for _ in range(_K - 1):
        t = jnp.max(jnp.where(sim >= t, _NEG, sim), axis=1, keepdims=True)

    e = jnp.where(sim >= t, jnp.exp(sim - m1), 0.0)
    z = jnp.sum(e, axis=1, keepdims=True)
    A = e / z
    A = jnp.where(row == col, 1.0, A)

    awc = aw_col_ref[0]
    t0 = t[0, 0]
    m0 = m1[0, 0]
    e0 = jnp.where(awc >= t0, jnp.exp(awc - m0), 0.0)
    r_col = e0 / jnp.sum(e0)
    rowc = lax.broadcasted_iota(jnp.int32, (N, 1), 0)
    a0c = jnp.where(rowc == 0, 1.0, r_col)
    A = jnp.where(col == 0, a0c, A)

    bf = jnp.bfloat16
    Ab = A.astype(bf)
    HE = jnp.dot(Ab, F.astype(bf), preferred_element_type=jnp.float32)
    h = jnp.dot(HE.astype(bf), edW_ref[...], preferred_element_type=jnp.float32) + edb_ref[...]
    h = jnp.where(h >= 0, h, 0.2 * h)
    HEr = jnp.dot(h.astype(bf), euW_ref[...], preferred_element_type=jnp.float32) + eub_ref[...]
    HC = lax.dot_general(Ab, HEr.astype(bf), (((0,), (0,)), ((), ())),
                         preferred_element_type=jnp.float32)
    h2 = jnp.dot(HC.astype(bf), ndW_ref[...], preferred_element_type=jnp.float32) + ndb_ref[...]
    h2 = jnp.where(h2 >= 0, h2, 0.2 * h2)
    out = jnp.dot(h2.astype(bf), nuW_ref[...], preferred_element_type=jnp.float32) + nub_ref[...]
    o_ref[0] = out


def _build_call(B, N, D, H, interpret=False):
    return pl.pallas_call(
        _body,
        grid=(B,),
        in_specs=[
            pl.BlockSpec((1, 1, N), lambda b: (b, 0, 0)),
            pl.BlockSpec((1, N, 1), lambda b: (b, 0, 0)),
            pl.BlockSpec((1, N, D), lambda b: (b, 0, 0)),
            pl.BlockSpec((D, H), lambda b: (0, 0)),
            pl.BlockSpec((1, H), lambda b: (0, 0)),
            pl.BlockSpec((H, D), lambda b: (0, 0)),
            pl.BlockSpec((1, D), lambda b: (0, 0)),
            pl.BlockSpec((D, H), lambda b: (0, 0)),
            pl.BlockSpec((1, H), lambda b: (0, 0)),
            pl.BlockSpec((H, D), lambda b: (0, 0)),
            pl.BlockSpec((1, D), lambda b: (0, 0)),
        ],
        out_specs=pl.BlockSpec((1, N, D), lambda b: (b, 0, 0)),
        out_shape=jax.ShapeDtypeStruct((B, N, D), jnp.float32),
        compiler_params=pltpu.CompilerParams(
            dimension_semantics=("arbitrary",)),
        interpret=interpret,
    )


def kernel(features, attn_weights, edge_down_W, edge_down_b, edge_up_W,
           edge_up_b, node_down_W, node_down_b, node_up_W, node_up_b):
    B, N, D = features.shape
    H = edge_down_W.shape[1]
    aw_pad = jnp.concatenate(
        [jnp.full((B, 1), _NEG, features.dtype), attn_weights], axis=1)
    call = _build_call(B, N, D, H)
    bf = jnp.bfloat16
    return call(aw_pad[:, None, :], aw_pad[:, :, None], features,
                edge_down_W.astype(bf), edge_down_b.reshape(1, -1),
                edge_up_W.astype(bf), edge_up_b.reshape(1, -1),
                node_down_W.astype(bf), node_down_b.reshape(1, -1),
                node_up_W.astype(bf), node_up_b.reshape(1, -1))
